# Initial kernel scaffold; baseline (speedup 1.0000x reference)
import jax
import jax.numpy as jnp
from jax.experimental import pallas as pl


def kernel(x, edge_index, W1, b1, W2, b2):
    raise NotImplementedError("write your pallas kernel here")


# speedup vs baseline: 3.2258x; 3.2258x over previous
import functools

import jax
import jax.numpy as jnp
from jax import lax
from jax.experimental import pallas as pl
from jax.experimental.pallas import tpu as pltpu
from jax.experimental.pallas import tpu_sc as plsc

N = 10000
NP = 10240
NH = NP // 2
NA = NH + 8
E = 320000
IN_DIM = 128
H1 = 64
H2 = 32

NC = 2
NS = 16
NW = NC * NS
CHUNK = 128
NCHUNKS = 2560
EP = NCHUNKS * CHUNK

BN = 1024
GRID = NP // BN


def _sc_mesh():
    return plsc.VectorSubcoreMesh(core_axis_name="c", subcore_axis_name="s")


@functools.partial(
    pl.kernel,
    mesh=_sc_mesh(),
    out_type=jax.ShapeDtypeStruct((NS, 2 * NH * 16), jnp.float32),
    scratch_types=[
        pltpu.VMEM((CHUNK,), jnp.int32),
        pltpu.VMEM((NA * 16,), jnp.float32),
    ],
)
def _deg_sc(lidx2d_hbm, deg_hbm, lidx_v, acc_v):
    c = lax.axis_index("c")
    s = lax.axis_index("s")
    w = c * NS + s
    h = w % 2
    q = w // 2
    cpt = NCHUNKS // NS
    e0 = jnp.where(lax.iota(jnp.int32, 16) == 0, 1.0, 0.0).astype(jnp.float32)
    z16 = jnp.zeros((16,), jnp.float32)

    def zero_body(i, carry):
        acc_v[pl.ds(16 * i, 16)] = z16
        return carry

    lax.fori_loop(0, NA, zero_body, 0)

    def chunk_body(j, carry):
        row = q * cpt + j
        pltpu.sync_copy(lidx2d_hbm.at[h, row], lidx_v)
        for k in range(CHUNK // 16):
            lv = lidx_v[pl.ds(16 * k, 16)]
            for e in range(16):
                plsc.addupdate(acc_v.at[pl.ds(lv[e] * 16, 16)], e0)
        return carry

    lax.fori_loop(0, cpt, chunk_body, 0)
    pltpu.sync_copy(acc_v.at[pl.ds(0, NH * 16)],
                    deg_hbm.at[q, pl.ds(h * (NH * 16), NH * 16)])


def _make_agg(nf, nq):
    cpt = NCHUNKS // nq

    @functools.partial(
        pl.kernel,
        mesh=_sc_mesh(),
        out_type=jax.ShapeDtypeStruct((nq, nf, 2 * NH * 16), jnp.float32),
        scratch_types=[
            pltpu.VMEM((CHUNK,), jnp.int32),
            pltpu.VMEM((CHUNK,), jnp.int32),
            pltpu.VMEM((CHUNK, 128), jnp.float32),
            pltpu.VMEM((NA * 16,), jnp.float32),
            pltpu.SemaphoreType.DMA,
        ],
    )
    def _agg(ypad_hbm, src2d_hbm, lidx2d_hbm, p_hbm,
             sidx_v, lidx_v, rows_v, acc_v, sem):
        c = lax.axis_index("c")
        s = lax.axis_index("s")
        w = c * NS + s
        f = w % nf
        h = (w // nf) % 2
        q = w // (2 * nf)
        f16 = f * 16
        z16 = jnp.zeros((16,), jnp.float32)

        def zero_body(i, carry):
            acc_v[pl.ds(16 * i, 16)] = z16
            return carry

        lax.fori_loop(0, NA, zero_body, 0)

        def chunk_body(j, carry):
            row = q * cpt + j
            pltpu.sync_copy(src2d_hbm.at[row], sidx_v)
            pltpu.sync_copy(lidx2d_hbm.at[h, row], lidx_v)
            pltpu.async_copy(ypad_hbm.at[sidx_v], rows_v, sem).wait()
            for k in range(CHUNK // 16):
                lv = lidx_v[pl.ds(16 * k, 16)]
                for e in range(16):
                    plsc.addupdate(acc_v.at[pl.ds(lv[e] * 16, 16)],
                                   rows_v[16 * k + e, pl.ds(f16, 16)])
            return carry

        lax.fori_loop(0, cpt, chunk_body, 0)
        pltpu.sync_copy(acc_v.at[pl.ds(0, NH * 16)],
                        p_hbm.at[q, f, pl.ds(h * (NH * 16), NH * 16)])

    return _agg


_agg_l1 = _make_agg(4, 4)
_agg_l2 = _make_agg(2, 8)


def _tc1_body(x_ref, w_ref, deg_ref, y_ref, dinv_ref):
    dd = deg_ref[...].reshape(NS, BN, 16)
    d = jnp.sum(dd[:, :, 0], axis=0) + 1.0
    dinv = lax.rsqrt(d)[:, None]
    xw = jnp.dot(x_ref[...], w_ref[...], preferred_element_type=jnp.float32)
    y_ref[...] = jnp.concatenate(
        [xw * dinv, jnp.zeros((BN, 128 - H1), jnp.float32)], axis=1)
    dinv_ref[...] = dinv


def _tc2_body(p_ref, y1_ref, dinv_ref, b1_ref, w2_ref, y2_ref):
    t = jnp.sum(p_ref[...].reshape(4, 4, BN, 16), axis=0)
    agg = jnp.concatenate([t[k] for k in range(4)], axis=1)
    y1 = y1_ref[:, :H1]
    dinv = dinv_ref[...]
    h = jnp.maximum(dinv * (agg + y1) + b1_ref[...], 0.0)
    y2 = jnp.dot(h, w2_ref[...], preferred_element_type=jnp.float32) * dinv
    y2_ref[...] = jnp.concatenate(
        [y2, jnp.zeros((BN, 128 - H2), jnp.float32)], axis=1)


def _tc3_body(p_ref, y2_ref, dinv_ref, b2_ref, o_ref):
    t = jnp.sum(p_ref[...].reshape(8, 2, BN, 16), axis=0)
    agg = jnp.concatenate([t[k] for k in range(2)], axis=1)
    y2 = y2_ref[:, :H2]
    o = dinv_ref[...] * (agg + y2) + b2_ref[...]
    o_ref[...] = jnp.maximum(o, 0.0)


def kernel(x, edge_index, W1, b1, W2, b2):
    src = edge_index[0].astype(jnp.int32)
    dst = edge_index[1].astype(jnp.int32)
    pad = N + (jnp.arange(EP - E, dtype=jnp.int32) % (NP - N))
    srcp = jnp.concatenate([src, pad])
    dstp = jnp.concatenate([dst, pad])
    src2d = srcp.reshape(NCHUNKS, CHUNK)
    half = jnp.arange(2, dtype=jnp.int32)[:, None]
    inh = (dstp[None, :] >= half * NH) & (dstp[None, :] < (half + 1) * NH)
    lidx2d = jnp.where(inh, dstp[None, :] - half * NH,
                       NH + (dstp[None, :] % 8)).reshape(2, NCHUNKS, CHUNK)
    x_pad = jnp.concatenate([x, jnp.zeros((NP - N, IN_DIM), jnp.float32)])

    deg = _deg_sc(lidx2d)

    y1, dinv = pl.pallas_call(
        _tc1_body,
        grid=(GRID,),
        in_specs=[
            pl.BlockSpec((BN, IN_DIM), lambda i: (i, 0)),
            pl.BlockSpec((IN_DIM, H1), lambda i: (0, 0)),
            pl.BlockSpec((NS, BN * 16), lambda i: (0, i)),
        ],
        out_specs=[
            pl.BlockSpec((BN, 128), lambda i: (i, 0)),
            pl.BlockSpec((BN, 1), lambda i: (i, 0)),
        ],
        out_shape=[
            jax.ShapeDtypeStruct((NP, 128), jnp.float32),
            jax.ShapeDtypeStruct((NP, 1), jnp.float32),
        ],
    )(x_pad, W1, deg)

    p1 = _agg_l1(y1, src2d, lidx2d)

    y2 = pl.pallas_call(
        _tc2_body,
        grid=(GRID,),
        in_specs=[
            pl.BlockSpec((4, 4, BN * 16), lambda i: (0, 0, i)),
            pl.BlockSpec((BN, 128), lambda i: (i, 0)),
            pl.BlockSpec((BN, 1), lambda i: (i, 0)),
            pl.BlockSpec((1, H1), lambda i: (0, 0)),
            pl.BlockSpec((H1, H2), lambda i: (0, 0)),
        ],
        out_specs=pl.BlockSpec((BN, 128), lambda i: (i, 0)),
        out_shape=jax.ShapeDtypeStruct((NP, 128), jnp.float32),
    )(p1, y1, dinv, b1.reshape(1, H1), W2)

    p2 = _agg_l2(y2, src2d, lidx2d)

    out = pl.pallas_call(
        _tc3_body,
        grid=(GRID,),
        in_specs=[
            pl.BlockSpec((8, 2, BN * 16), lambda i: (0, 0, i)),
            pl.BlockSpec((BN, 128), lambda i: (i, 0)),
            pl.BlockSpec((BN, 1), lambda i: (i, 0)),
            pl.BlockSpec((1, H2), lambda i: (0, 0)),
        ],
        out_specs=pl.BlockSpec((BN, H2), lambda i: (i, 0)),
        out_shape=jax.ShapeDtypeStruct((NP, H2), jnp.float32),
    )(p2, y2, dinv, b2.reshape(1, H2))

    return out[:N]


# speedup vs baseline: 4.4154x; 1.3688x over previous
import functools

import jax
import jax.numpy as jnp
from jax import lax
from jax.experimental import pallas as pl
from jax.experimental.pallas import tpu as pltpu
from jax.experimental.pallas import tpu_sc as plsc

N = 10000
NP = 10240
NH = NP // 2
NA = NH + 8
E = 320000
IN_DIM = 128
H1 = 64
H2 = 32

NC = 2
NS = 16
CHUNK = 128
NCHUNKS = 2560
EP = NCHUNKS * CHUNK

BN = 1024
GRID = NP // BN


def _sc_mesh():
    return plsc.VectorSubcoreMesh(core_axis_name="c", subcore_axis_name="s")


@functools.partial(
    pl.kernel,
    mesh=_sc_mesh(),
    out_type=jax.ShapeDtypeStruct((NS, 2 * NH * 16), jnp.float32),
    scratch_types=[
        pltpu.VMEM((CHUNK,), jnp.int32),
        pltpu.VMEM((NA * 16,), jnp.float32),
    ],
)
def _deg_sc(lidx2d_hbm, deg_hbm, lidx_v, acc_v):
    c = lax.axis_index("c")
    s = lax.axis_index("s")
    w = c * NS + s
    h = w % 2
    q = w // 2
    cpt = NCHUNKS // NS
    e0 = jnp.where(lax.iota(jnp.int32, 16) == 0, 1.0, 0.0).astype(jnp.float32)
    z16 = jnp.zeros((16,), jnp.float32)

    def zero_body(i, carry):
        acc_v[pl.ds(16 * i, 16)] = z16
        return carry

    lax.fori_loop(0, NA, zero_body, 0)

    def chunk_body(j, carry):
        row = q * cpt + j
        pltpu.sync_copy(lidx2d_hbm.at[h, row], lidx_v)
        for k in range(CHUNK // 16):
            lv = lidx_v[pl.ds(16 * k, 16)]
            for e in range(16):
                plsc.addupdate(acc_v.at[pl.ds(lv[e] * 16, 16)], e0)
        return carry

    lax.fori_loop(0, cpt, chunk_body, 0)
    pltpu.sync_copy(acc_v.at[pl.ds(0, NH * 16)],
                    deg_hbm.at[q, pl.ds(h * (NH * 16), NH * 16)])


def _make_agg(nf, nq):
    cpt = NCHUNKS // nq

    @functools.partial(
        pl.kernel,
        mesh=_sc_mesh(),
        out_type=jax.ShapeDtypeStruct((nq, nf, 2 * NH * 16), jnp.float32),
        scratch_types=[
            pltpu.VMEM((CHUNK,), jnp.int32),
            pltpu.VMEM((CHUNK,), jnp.int32),
            pltpu.VMEM((CHUNK,), jnp.int32),
            pltpu.VMEM((CHUNK,), jnp.int32),
            pltpu.VMEM((CHUNK, 128), jnp.float32),
            pltpu.VMEM((CHUNK, 128), jnp.float32),
            pltpu.VMEM((NA * 16,), jnp.float32),
            pltpu.SemaphoreType.DMA,
            pltpu.SemaphoreType.DMA,
        ],
    )
    def _agg(ypad_hbm, src2d_hbm, lidx2d_hbm, p_hbm,
             sidx_a, sidx_b, lidx_a, lidx_b, rows_a, rows_b, acc_v,
             sem_a, sem_b):
        c = lax.axis_index("c")
        s = lax.axis_index("s")
        w = c * NS + s
        f = w % nf
        h = (w // nf) % 2
        q = w // (2 * nf)
        f16 = f * 16
        z16 = jnp.zeros((16,), jnp.float32)

        def zero_body(i, carry):
            acc_v[pl.ds(16 * i, 16)] = z16
            return carry

        lax.fori_loop(0, NA, zero_body, 0)

        def accumulate(lidx_v, rows_v):
            for k in range(CHUNK // 16):
                lv = lidx_v[pl.ds(16 * k, 16)]
                for e in range(16):
                    plsc.addupdate(acc_v.at[pl.ds(lv[e] * 16, 16)],
                                   rows_v[16 * k + e, pl.ds(f16, 16)])

        def pair_body(jj, carry):
            r0 = q * cpt + 2 * jj
            pltpu.sync_copy(src2d_hbm.at[r0], sidx_a)
            pltpu.sync_copy(lidx2d_hbm.at[h, r0], lidx_a)
            ha = pltpu.async_copy(ypad_hbm.at[sidx_a], rows_a, sem_a)
            pltpu.sync_copy(src2d_hbm.at[r0 + 1], sidx_b)
            pltpu.sync_copy(lidx2d_hbm.at[h, r0 + 1], lidx_b)
            hb = pltpu.async_copy(ypad_hbm.at[sidx_b], rows_b, sem_b)
            ha.wait()
            accumulate(lidx_a, rows_a)
            hb.wait()
            accumulate(lidx_b, rows_b)
            return carry

        lax.fori_loop(0, cpt // 2, pair_body, 0)
        pltpu.sync_copy(acc_v.at[pl.ds(0, NH * 16)],
                        p_hbm.at[q, f, pl.ds(h * (NH * 16), NH * 16)])

    return _agg


_agg_l1 = _make_agg(4, 4)
_agg_l2 = _make_agg(2, 8)


def _tc1_body(x_ref, w_ref, deg_ref, y_ref, dinv_ref):
    dd = deg_ref[...].reshape(NS, BN, 16)
    d = jnp.sum(dd[:, :, 0], axis=0) + 1.0
    dinv = lax.rsqrt(d)[:, None]
    xw = jnp.dot(x_ref[...], w_ref[...], preferred_element_type=jnp.float32)
    y_ref[...] = jnp.concatenate(
        [xw * dinv, jnp.zeros((BN, 128 - H1), jnp.float32)], axis=1)
    dinv_ref[...] = dinv


def _tc2_body(p_ref, y1_ref, dinv_ref, b1_ref, w2_ref, y2_ref):
    t = jnp.sum(p_ref[...].reshape(4, 4, BN, 16), axis=0)
    agg = jnp.concatenate([t[k] for k in range(4)], axis=1)
    y1 = y1_ref[:, :H1]
    dinv = dinv_ref[...]
    h = jnp.maximum(dinv * (agg + y1) + b1_ref[...], 0.0)
    y2 = jnp.dot(h, w2_ref[...], preferred_element_type=jnp.float32) * dinv
    y2_ref[...] = jnp.concatenate(
        [y2, jnp.zeros((BN, 128 - H2), jnp.float32)], axis=1)


def _tc3_body(p_ref, y2_ref, dinv_ref, b2_ref, o_ref):
    t = jnp.sum(p_ref[...].reshape(8, 2, BN, 16), axis=0)
    agg = jnp.concatenate([t[k] for k in range(2)], axis=1)
    y2 = y2_ref[:, :H2]
    o = dinv_ref[...] * (agg + y2) + b2_ref[...]
    o_ref[...] = jnp.maximum(o, 0.0)


def kernel(x, edge_index, W1, b1, W2, b2):
    src = edge_index[0].astype(jnp.int32)
    dst = edge_index[1].astype(jnp.int32)
    pad = N + (jnp.arange(EP - E, dtype=jnp.int32) % (NP - N))
    srcp = jnp.concatenate([src, pad])
    dstp = jnp.concatenate([dst, pad])
    src2d = srcp.reshape(NCHUNKS, CHUNK)
    half = jnp.arange(2, dtype=jnp.int32)[:, None]
    inh = (dstp[None, :] >= half * NH) & (dstp[None, :] < (half + 1) * NH)
    lidx2d = jnp.where(inh, dstp[None, :] - half * NH,
                       NH + (dstp[None, :] % 8)).reshape(2, NCHUNKS, CHUNK)
    x_pad = jnp.concatenate([x, jnp.zeros((NP - N, IN_DIM), jnp.float32)])

    deg = _deg_sc(lidx2d)

    y1, dinv = pl.pallas_call(
        _tc1_body,
        grid=(GRID,),
        in_specs=[
            pl.BlockSpec((BN, IN_DIM), lambda i: (i, 0)),
            pl.BlockSpec((IN_DIM, H1), lambda i: (0, 0)),
            pl.BlockSpec((NS, BN * 16), lambda i: (0, i)),
        ],
        out_specs=[
            pl.BlockSpec((BN, 128), lambda i: (i, 0)),
            pl.BlockSpec((BN, 1), lambda i: (i, 0)),
        ],
        out_shape=[
            jax.ShapeDtypeStruct((NP, 128), jnp.float32),
            jax.ShapeDtypeStruct((NP, 1), jnp.float32),
        ],
    )(x_pad, W1, deg)

    p1 = _agg_l1(y1, src2d, lidx2d)

    y2 = pl.pallas_call(
        _tc2_body,
        grid=(GRID,),
        in_specs=[
            pl.BlockSpec((4, 4, BN * 16), lambda i: (0, 0, i)),
            pl.BlockSpec((BN, 128), lambda i: (i, 0)),
            pl.BlockSpec((BN, 1), lambda i: (i, 0)),
            pl.BlockSpec((1, H1), lambda i: (0, 0)),
            pl.BlockSpec((H1, H2), lambda i: (0, 0)),
        ],
        out_specs=pl.BlockSpec((BN, 128), lambda i: (i, 0)),
        out_shape=jax.ShapeDtypeStruct((NP, 128), jnp.float32),
    )(p1, y1, dinv, b1.reshape(1, H1), W2)

    p2 = _agg_l2(y2, src2d, lidx2d)

    out = pl.pallas_call(
        _tc3_body,
        grid=(GRID,),
        in_specs=[
            pl.BlockSpec((8, 2, BN * 16), lambda i: (0, 0, i)),
            pl.BlockSpec((BN, 128), lambda i: (i, 0)),
            pl.BlockSpec((BN, 1), lambda i: (i, 0)),
            pl.BlockSpec((1, H2), lambda i: (0, 0)),
        ],
        out_specs=pl.BlockSpec((BN, H2), lambda i: (i, 0)),
        out_shape=jax.ShapeDtypeStruct((NP, H2), jnp.float32),
    )(p2, y2, dinv, b2.reshape(1, H2))

    return out[:N]
